# Initial kernel scaffold; baseline (speedup 1.0000x reference)
import jax
import jax.numpy as jnp
from jax.experimental import pallas as pl


def kernel(input_hr, input_lr, embed_lr, embed_hr):
    raise NotImplementedError("write your pallas kernel here")


# speedup vs baseline: 1.5161x; 1.5161x over previous
import functools

import jax
import jax.numpy as jnp
from jax import lax
from jax.experimental import pallas as pl
from jax.experimental.pallas import tpu as pltpu
from jax.experimental.pallas import tpu_sc as plsc

_DIM = 256
_K = 8192
_NTOK = 2048
_TT = 256
_GRID = _NTOK // _TT

_SC_CORES = 2
_SC_SUBCORES = 16
_NW = _SC_CORES * _SC_SUBCORES
_B = 2 * _NTOK
_BPW = _B // _NW


def _dist_body(xh_ref, xl_ref, e_ref, dh_ref, dl_ref, ih_ref, il_ref, ms_ref):
    step = pl.program_id(0)
    e = e_ref[...]
    ee = jnp.sum(e * e, axis=0, keepdims=True)

    @pl.when(step == 0)
    def _():
        ms_ref[0, 0] = 0.0
        ms_ref[0, 1] = 0.0

    def one(x_ref, d_ref, i_ref, slot):
        x = x_ref[...]
        xx = jnp.sum(x * x, axis=1, keepdims=True)
        xe = jnp.dot(x, e, preferred_element_type=jnp.float32)
        dist = (xx - 2.0 * xe) + ee
        d_ref[...] = dist
        m = jnp.min(dist, axis=1, keepdims=True)
        iota = lax.broadcasted_iota(jnp.int32, dist.shape, 1)
        ind = jnp.min(jnp.where(dist == m, iota, _K), axis=1)
        i_ref[...] = ind.reshape(1, 1, _TT)
        ms_ref[0, slot] += jnp.sum(m)

    one(xh_ref, dh_ref, ih_ref, 0)
    one(xl_ref, dl_ref, il_ref, 1)


def _dist_call(xh, xl, e):
    return pl.pallas_call(
        _dist_body,
        grid=(_GRID,),
        in_specs=[
            pl.BlockSpec((_TT, _DIM), lambda i: (i, 0)),
            pl.BlockSpec((_TT, _DIM), lambda i: (i, 0)),
            pl.BlockSpec((_DIM, _K), lambda i: (0, 0)),
        ],
        out_specs=[
            pl.BlockSpec((_TT, _K), lambda i: (i, 0)),
            pl.BlockSpec((_TT, _K), lambda i: (i, 0)),
            pl.BlockSpec((1, 1, _TT), lambda i: (i, 0, 0)),
            pl.BlockSpec((1, 1, _TT), lambda i: (i, 0, 0)),
            pl.BlockSpec((1, 2), lambda i: (0, 0), memory_space=pltpu.SMEM),
        ],
        out_shape=[
            jax.ShapeDtypeStruct((_NTOK, _K), jnp.float32),
            jax.ShapeDtypeStruct((_NTOK, _K), jnp.float32),
            jax.ShapeDtypeStruct((_GRID, 1, _TT), jnp.int32),
            jax.ShapeDtypeStruct((_GRID, 1, _TT), jnp.int32),
            jax.ShapeDtypeStruct((1, 2), jnp.float32),
        ],
    )(xh, xl, e)


def _gather_call(table, idx):
    mesh = plsc.VectorSubcoreMesh(core_axis_name="c", subcore_axis_name="s")

    @functools.partial(
        pl.kernel,
        mesh=mesh,
        out_type=jax.ShapeDtypeStruct((_B, _DIM), jnp.float32),
        scratch_types=[
            pltpu.VMEM((_BPW,), jnp.int32),
            pltpu.VMEM((_BPW, _DIM), jnp.float32),
            pltpu.SemaphoreType.DMA,
        ],
    )
    def gk(table_hbm, idx_hbm, out_hbm, idx_v, rows_v, sem):
        wid = lax.axis_index("s") * _SC_CORES + lax.axis_index("c")
        base = wid * _BPW
        pltpu.sync_copy(idx_hbm.at[pl.ds(base, _BPW)], idx_v)
        pltpu.async_copy(table_hbm.at[idx_v], rows_v, sem).wait()
        pltpu.sync_copy(rows_v, out_hbm.at[pl.ds(base, _BPW)])

    return gk(table, idx)


def kernel(input_hr, input_lr, embed_lr, embed_hr):
    xh = input_hr.reshape(-1, _DIM)
    xl = input_lr.reshape(-1, _DIM)
    dist_h, dist_l, ih, il, ms = _dist_call(xh, xl, embed_lr)
    ind_h = ih.reshape(_NTOK)
    ind_l = il.reshape(_NTOK)
    idx_all = jnp.concatenate([ind_h, ind_l])
    q = _gather_call(embed_lr.T, idx_all)
    q_h = q[:_NTOK].reshape(input_hr.shape)
    q_l = q[_NTOK:].reshape(input_lr.shape)
    diff_h = ms[0, 0] / (_NTOK * _DIM)
    diff_l = ms[0, 1] / (_NTOK * _DIM)
    ei_h = ind_h.reshape(input_hr.shape[:-1])
    ei_l = ind_l.reshape(input_lr.shape[:-1])
    return (q_h, q_l, q_h, q_l,
            diff_h, diff_l, diff_h, diff_l,
            ei_h, ei_l, ei_h, ei_l,
            dist_h, dist_l, dist_h, dist_l)


# speedup vs baseline: 2.3124x; 1.5253x over previous
import functools

import jax
import jax.numpy as jnp
from jax import lax
from jax.experimental import pallas as pl
from jax.experimental.pallas import tpu as pltpu
from jax.experimental.pallas import tpu_sc as plsc

_DIM = 256
_K = 8192
_NTOK = 2048
_TT = 128
_GRID = _NTOK // _TT

_SC_CORES = 2
_SC_SUBCORES = 16
_NW = _SC_CORES * _SC_SUBCORES
_BPW = 2 * _NTOK // _NW


def _dist_body(xh_ref, xl_ref, e_ref,
               dh1_ref, dh2_ref, dl1_ref, dl2_ref, ih_ref, il_ref, ms_ref):
    step = pl.program_id(0)
    e = e_ref[...]
    ee = jnp.sum(e * e, axis=0, keepdims=True)

    @pl.when(step == 0)
    def _():
        ms_ref[0, 0] = 0.0
        ms_ref[0, 1] = 0.0

    def one(x_ref, d1_ref, d2_ref, i_ref, slot):
        x = x_ref[...]
        xx = jnp.sum(x * x, axis=1, keepdims=True)
        xe = jnp.dot(x, e, preferred_element_type=jnp.float32)
        dist = (xx - 2.0 * xe) + ee
        d1_ref[...] = dist
        d2_ref[...] = dist
        m = jnp.min(dist, axis=1, keepdims=True)
        iota = lax.broadcasted_iota(jnp.int32, dist.shape, 1)
        ind = jnp.min(jnp.where(dist == m, iota, _K), axis=1)
        i_ref[...] = ind.reshape(1, 1, _TT)
        ms_ref[0, slot] += jnp.sum(m)

    one(xh_ref, dh1_ref, dh2_ref, ih_ref, 0)
    one(xl_ref, dl1_ref, dl2_ref, il_ref, 1)


def _dist_call(xh, xl, e):
    dspec = pl.BlockSpec((_TT, _K), lambda i: (i, 0))
    ispec = pl.BlockSpec((1, 1, _TT), lambda i: (i, 0, 0))
    dshape = jax.ShapeDtypeStruct((_NTOK, _K), jnp.float32)
    ishape = jax.ShapeDtypeStruct((_GRID, 1, _TT), jnp.int32)
    return pl.pallas_call(
        _dist_body,
        grid=(_GRID,),
        in_specs=[
            pl.BlockSpec((_TT, _DIM), lambda i: (i, 0)),
            pl.BlockSpec((_TT, _DIM), lambda i: (i, 0)),
            pl.BlockSpec((_DIM, _K), lambda i: (0, 0)),
        ],
        out_specs=[
            dspec, dspec, dspec, dspec, ispec, ispec,
            pl.BlockSpec((1, 2), lambda i: (0, 0), memory_space=pltpu.SMEM),
        ],
        out_shape=[
            dshape, dshape, dshape, dshape, ishape, ishape,
            jax.ShapeDtypeStruct((1, 2), jnp.float32),
        ],
    )(xh, xl, e)


def _gather_call(table, idx_h, idx_l):
    mesh = plsc.VectorSubcoreMesh(core_axis_name="c", subcore_axis_name="s")
    qshape = jax.ShapeDtypeStruct((_NTOK, _DIM), jnp.float32)

    @functools.partial(
        pl.kernel,
        mesh=mesh,
        out_type=(qshape, qshape, qshape, qshape),
        scratch_types=[
            pltpu.VMEM((_BPW,), jnp.int32),
            pltpu.VMEM((_BPW, _DIM), jnp.float32),
            pltpu.SemaphoreType.DMA,
        ],
    )
    def gk(table_hbm, ih_hbm, il_hbm, qh1_hbm, qh2_hbm, ql1_hbm, ql2_hbm,
           idx_v, rows_v, sem):
        wid = lax.axis_index("s") * _SC_CORES + lax.axis_index("c")
        is_lr = wid >= _NW // 2
        base = jnp.where(is_lr, (wid - _NW // 2) * _BPW, wid * _BPW)

        @pl.when(jnp.logical_not(is_lr))
        def _():
            pltpu.sync_copy(ih_hbm.at[pl.ds(base, _BPW)], idx_v)
            pltpu.async_copy(table_hbm.at[idx_v], rows_v, sem).wait()
            pltpu.sync_copy(rows_v, qh1_hbm.at[pl.ds(base, _BPW)])
            pltpu.sync_copy(rows_v, qh2_hbm.at[pl.ds(base, _BPW)])

        @pl.when(is_lr)
        def _():
            pltpu.sync_copy(il_hbm.at[pl.ds(base, _BPW)], idx_v)
            pltpu.async_copy(table_hbm.at[idx_v], rows_v, sem).wait()
            pltpu.sync_copy(rows_v, ql1_hbm.at[pl.ds(base, _BPW)])
            pltpu.sync_copy(rows_v, ql2_hbm.at[pl.ds(base, _BPW)])

    return gk(table, idx_h, idx_l)


def kernel(input_hr, input_lr, embed_lr, embed_hr):
    xh = input_hr.reshape(-1, _DIM)
    xl = input_lr.reshape(-1, _DIM)
    dh1, dh2, dl1, dl2, ih, il, ms = _dist_call(xh, xl, embed_lr)
    ind_h = ih.reshape(_NTOK)
    ind_l = il.reshape(_NTOK)
    qh1, qh2, ql1, ql2 = _gather_call(embed_lr.T, ind_h, ind_l)
    q_h1 = qh1.reshape(input_hr.shape)
    q_h2 = qh2.reshape(input_hr.shape)
    q_l1 = ql1.reshape(input_lr.shape)
    q_l2 = ql2.reshape(input_lr.shape)
    diff_h = ms[0, 0] / (_NTOK * _DIM)
    diff_l = ms[0, 1] / (_NTOK * _DIM)
    ei_h = ind_h.reshape(input_hr.shape[:-1])
    ei_l = ind_l.reshape(input_lr.shape[:-1])
    return (q_h1, q_l1, q_h2, q_l2,
            diff_h, diff_l, diff_h, diff_l,
            ei_h, ei_l, ei_h, ei_l,
            dh1, dl1, dh2, dl2)
